# baseline (speedup 1.0000x reference)
import functools

import jax
import jax.numpy as jnp
from jax import lax
from jax.experimental import pallas as pl
from jax.experimental.pallas import tpu as pltpu
from jax.experimental.pallas import tpu_sc as plsc

D = 64
V = 1000000
B, H = 4096, 50
N = B * H

BLK_I = 32768
HB = BLK_I // 2
GRID = (V + BLK_I - 1) // BLK_I
W2_ROWS = GRID * HB


def _relayout_body(wt_ref, o_ref):
    x = wt_ref[...]
    o_ref[...] = jnp.concatenate([x[:, :HB].T, x[:, HB:].T], axis=1)


_relayout = pl.pallas_call(
    _relayout_body,
    grid=(GRID,),
    in_specs=[pl.BlockSpec((D, BLK_I), lambda i: (0, i))],
    out_specs=pl.BlockSpec((HB, 2 * D), lambda i: (i, 0)),
    out_shape=jax.ShapeDtypeStruct((W2_ROWS, 2 * D), jnp.float32),
)

NC, NS = 2, 16
NW = NC * NS
PER_W = N // NW
CHUNK = 400
NBUF = 2
NCHUNK = PER_W // CHUNK
ROUNDS = NCHUNK // NBUF

_MESH = plsc.VectorSubcoreMesh(core_axis_name="c", subcore_axis_name="s")


@functools.partial(
    pl.kernel,
    mesh=_MESH,
    out_type=jax.ShapeDtypeStruct((N, D), jnp.float32),
    scratch_types=[
        pltpu.VMEM((PER_W,), jnp.int32),
        pltpu.VMEM((NBUF, CHUNK, D), jnp.float32),
        pltpu.SemaphoreType.DMA((NBUF,)),
        pltpu.SemaphoreType.DMA((NBUF,)),
    ],
    compiler_params=pltpu.CompilerParams(use_tc_tiling_on_sc=False),
)
def _gather_kernel(idx_hbm, table_hbm, out_hbm, idx_v, rows, sem_g, sem_w):
    wid = lax.axis_index("s") * NC + lax.axis_index("c")
    base = wid * PER_W
    pltpu.sync_copy(idx_hbm.at[pl.ds(base, PER_W)], idx_v)

    def gather(b, c):
        return pltpu.make_async_copy(
            table_hbm.at[idx_v.at[pl.ds(c * CHUNK, CHUNK)]],
            rows.at[b], sem_g.at[b])

    def write(b, c):
        return pltpu.make_async_copy(
            rows.at[b], out_hbm.at[pl.ds(base + c * CHUNK, CHUNK)],
            sem_w.at[b])

    for b in range(NBUF):
        gather(b, b).start()

    def round_body(r, carry):
        c0 = r * NBUF
        for b in range(NBUF):
            gather(b, c0 + b).wait()
            write(b, c0 + b).start()
        for b in range(NBUF):
            write(b, c0 + b).wait()
            gather(b, c0 + NBUF + b).start()
        return carry

    lax.fori_loop(0, ROUNDS - 1, round_body, 0)

    c0 = (ROUNDS - 1) * NBUF
    for b in range(NBUF):
        gather(b, c0 + b).wait()
        write(b, c0 + b).start()
    for b in range(NBUF):
        write(b, c0 + b).wait()


def _select_body(pairs_ref, o_ref):
    xt = pairs_ref[...].T
    o_ref[...] = jnp.concatenate([xt[:D, :], xt[D:, :]], axis=1)[None]


_select_t = pl.pallas_call(
    _select_body,
    grid=(H,),
    in_specs=[pl.BlockSpec((B // 2, 2 * D), lambda h: (h, 0))],
    out_specs=pl.BlockSpec((1, D, B), lambda h: (h, 0, 0)),
    out_shape=jax.ShapeDtypeStruct((H, D, B), jnp.float32),
)


def kernel(input, weight):
    flat = input.reshape(2, B // 2, H).transpose(2, 1, 0).reshape(-1)
    flat = flat.astype(jnp.int32)
    j = flat >> 15
    local = flat & (BLK_I - 1)
    idx64 = (j << 15) + ((local & (HB - 1)) << 1) + (local >> 14)
    w2 = _relayout(weight.T)
    w64 = w2.reshape(2 * W2_ROWS, D)
    rows = _gather_kernel(idx64, w64)
    out_t = _select_t(rows.reshape(N // 2, 2 * D))
    return jnp.transpose(out_t, (2, 0, 1))
